# baseline (speedup 1.0000x reference)
import functools

import jax
import jax.numpy as jnp
from jax import lax
from jax.experimental import pallas as pl
from jax.experimental.pallas import tpu as pltpu
from jax.experimental.pallas import tpu_sc as plsc

TBP = 128
NC, NS, L = 2, 16, 16
NW = NC * NS


def _proj_kernel(lens_ref, f_ref, g_ref, w_ref, bias_ref,
                 ff_ref, gg_ref, tm_ref, um_ref, *, H1, B):
    ti = pl.program_id(0)
    wf = w_ref[:H1, :]
    for b in range(B):
        f_len = lens_ref[0, b]
        ff_ref[b] = (
            jnp.dot(f_ref[:, b, :], wf, preferred_element_type=jnp.float32)
            + bias_ref[0]
        )
        t_ids = ti * TBP + jax.lax.broadcasted_iota(jnp.int32, (TBP, L), 0)
        tm_ref[b] = (t_ids < f_len).astype(jnp.float32)

    @pl.when(ti == 0)
    def _():
        wg = w_ref[H1:, :]
        for b in range(B):
            g_len = lens_ref[1, b]
            gg_ref[b] = jnp.dot(g_ref[b], wg, preferred_element_type=jnp.float32)
            u_ids = jax.lax.broadcasted_iota(jnp.int32, (gg_ref.shape[1], L), 0)
            um_ref[b] = (u_ids < g_len).astype(jnp.float32)


def _make_sc_add(B, T, U, V):
    TPW = (B * T) // NW
    WPB = T // TPW
    mesh = plsc.VectorSubcoreMesh(core_axis_name="c", subcore_axis_name="s")

    @functools.partial(
        pl.kernel, mesh=mesh,
        out_type=jax.ShapeDtypeStruct((B, T, U, V), jnp.float32),
        scratch_types=[
            pltpu.VMEM((TPW, V), jnp.float32),
            pltpu.VMEM((U, V), jnp.float32),
            pltpu.VMEM((TPW, L), jnp.float32),
            pltpu.VMEM((U, L), jnp.float32),
            pltpu.VMEM((2, U, V), jnp.float32),
            pltpu.SemaphoreType.DMA((2,)),
        ],
    )
    def sc_add(ff_hbm, gg_hbm, tm_hbm, um_hbm, out_hbm,
               ffv, ggv, tmv, umv, slab, sem):
        wid = lax.axis_index("s") * NC + lax.axis_index("c")
        b = wid // WPB
        t0 = (wid % WPB) * TPW

        pltpu.sync_copy(ff_hbm.at[b, pl.ds(t0, TPW)], ffv)
        pltpu.sync_copy(gg_hbm.at[b], ggv)
        pltpu.sync_copy(tm_hbm.at[b, pl.ds(t0, TPW)], tmv)
        pltpu.sync_copy(um_hbm.at[b], umv)

        def slab_copy(i, sl):
            return pltpu.make_async_copy(
                slab.at[sl], out_hbm.at[b, t0 + i], sem.at[sl])

        def body(i, carry):
            sl = lax.rem(i, 2)

            @pl.when(i >= 2)
            def _():
                slab_copy(i - 2, sl).wait()

            t16 = tmv[i]
            for u in range(U):
                m16 = t16 * umv[u]
                for k in range(V // L):
                    s = ffv[i, pl.ds(k * L, L)] + ggv[u, pl.ds(k * L, L)]
                    slab[sl, u, pl.ds(k * L, L)] = s * m16
            slab_copy(i, sl).start()
            return carry

        lax.fori_loop(0, TPW, body, 0)
        slab_copy(TPW - 2, lax.rem(TPW - 2, 2)).wait()
        slab_copy(TPW - 1, lax.rem(TPW - 1, 2)).wait()

    return sc_add


def kernel(f, f_lens, g, g_lens, W, b):
    T, B, H1 = f.shape
    _, U, H2 = g.shape
    V = W.shape[1]

    lens = jnp.stack([f_lens, g_lens]).astype(jnp.int32)
    bias2d = b.reshape(1, V)

    ff, gg, tm, um = pl.pallas_call(
        functools.partial(_proj_kernel, H1=H1, B=B),
        grid_spec=pltpu.PrefetchScalarGridSpec(
            num_scalar_prefetch=1,
            grid=(T // TBP,),
            in_specs=[
                pl.BlockSpec((TBP, B, H1), lambda ti, lens: (ti, 0, 0)),
                pl.BlockSpec((B, U, H2), lambda ti, lens: (0, 0, 0)),
                pl.BlockSpec((H1 + H2, V), lambda ti, lens: (0, 0)),
                pl.BlockSpec((1, V), lambda ti, lens: (0, 0)),
            ],
            out_specs=[
                pl.BlockSpec((B, TBP, V), lambda ti, lens: (0, ti, 0)),
                pl.BlockSpec((B, U, V), lambda ti, lens: (0, 0, 0)),
                pl.BlockSpec((B, TBP, L), lambda ti, lens: (0, ti, 0)),
                pl.BlockSpec((B, U, L), lambda ti, lens: (0, 0, 0)),
            ],
        ),
        out_shape=[
            jax.ShapeDtypeStruct((B, T, V), jnp.float32),
            jax.ShapeDtypeStruct((B, U, V), jnp.float32),
            jax.ShapeDtypeStruct((B, T, L), jnp.float32),
            jax.ShapeDtypeStruct((B, U, L), jnp.float32),
        ],
    )(lens, f, g, W, bias2d)

    out = _make_sc_add(B, T, U, V)(ff, gg, tm, um)
    return (out, f_lens)


# speedup vs baseline: 12.7349x; 12.7349x over previous
import functools

import jax
import jax.numpy as jnp
from jax.experimental import pallas as pl
from jax.experimental.pallas import tpu as pltpu

TB = 256
NBUF = 3


def _joint_kernel(lens_ref, f_ref, g_ref, w_ref, bias_ref, out_hbm,
                  ff_vmem, gg_vmem, out_vmem, sems, *, H1, B, NT):
    ti = pl.program_id(0)
    j = pl.program_id(1)

    def out_copy(s_b, s_ti, s):
        return pltpu.make_async_copy(
            out_vmem.at[s],
            out_hbm.at[s_b, pl.ds(s_ti * TB, TB)],
            sems.at[s],
        )

    @pl.when(j == 0)
    def _proj():
        wf = w_ref[:H1, :]
        for b in range(B):
            ff_vmem[b] = (
                jnp.dot(f_ref[:, b, :], wf, preferred_element_type=jnp.float32)
                + bias_ref[0]
            )

        @pl.when(ti == 0)
        def _():
            wg = w_ref[H1:, :]
            for b in range(B):
                gg_vmem[b] = jnp.dot(
                    g_ref[b], wg, preferred_element_type=jnp.float32)

    @pl.when(j > 0)
    def _add():
        b = j - 1
        astep = ti * B + b
        slot = jax.lax.rem(astep, NBUF)

        @pl.when(astep >= NBUF)
        def _():
            prev = astep - NBUF
            out_copy(jax.lax.rem(prev, B), prev // B, slot).wait()

        f_len = lens_ref[0, b]
        g_len = lens_ref[1, b]

        ff = ff_vmem[b]
        gg = gg_vmem[b]
        U, V = gg.shape

        t_ids = ti * TB + jax.lax.broadcasted_iota(jnp.int32, (TB, V), 0)
        u_ids = jax.lax.broadcasted_iota(jnp.int32, (U, V), 0)
        tmask = (t_ids < f_len).astype(jnp.float32)
        umask = (u_ids < g_len).astype(jnp.float32)

        summed = ff[:, None, :] + gg[None, :, :]
        out_vmem[slot] = summed * tmask[:, None, :] * umask[None, :, :]

        out_copy(b, ti, slot).start()

        @pl.when(astep == NT * B - 1)
        def _():
            for k in range(NBUF - 1, -1, -1):
                prev = NT * B - 1 - k
                s = jax.lax.rem(prev, NBUF)
                out_copy(jax.lax.rem(prev, B), prev // B, s).wait()


def kernel(f, f_lens, g, g_lens, W, b):
    T, B, H1 = f.shape
    _, U, H2 = g.shape
    V = W.shape[1]
    NT = T // TB

    lens = jnp.stack([f_lens, g_lens]).astype(jnp.int32)
    bias2d = b.reshape(1, V)

    out = pl.pallas_call(
        functools.partial(_joint_kernel, H1=H1, B=B, NT=NT),
        grid_spec=pltpu.PrefetchScalarGridSpec(
            num_scalar_prefetch=1,
            grid=(NT, 1 + B),
            in_specs=[
                pl.BlockSpec((TB, B, H1), lambda ti, j, lens: (ti, 0, 0)),
                pl.BlockSpec((B, U, H2), lambda ti, j, lens: (0, 0, 0)),
                pl.BlockSpec((H1 + H2, V), lambda ti, j, lens: (0, 0)),
                pl.BlockSpec((1, V), lambda ti, j, lens: (0, 0)),
            ],
            out_specs=pl.BlockSpec(memory_space=pl.ANY),
            scratch_shapes=[
                pltpu.VMEM((B, TB, V), jnp.float32),
                pltpu.VMEM((B, U, V), jnp.float32),
                pltpu.VMEM((NBUF, TB, U, V), jnp.float32),
                pltpu.SemaphoreType.DMA((NBUF,)),
            ],
        ),
        out_shape=jax.ShapeDtypeStruct((B, T, U, V), jnp.float32),
    )(lens, f, g, W, bias2d)
    return (out, f_lens)


# speedup vs baseline: 12.8188x; 1.0066x over previous
import functools

import jax
import jax.numpy as jnp
from jax.experimental import pallas as pl
from jax.experimental.pallas import tpu as pltpu

TB = 256
NBUF = 4


def _joint_kernel(lens_ref, f_ref, g_ref, w_ref, bias_ref, out_hbm,
                  ff_vmem, gg_vmem, out_vmem, sems, *, H1, B, NT):
    ti = pl.program_id(0)
    j = pl.program_id(1)

    def out_copy(s_b, s_ti, s):
        return pltpu.make_async_copy(
            out_vmem.at[s],
            out_hbm.at[s_b, pl.ds(s_ti * TB, TB)],
            sems.at[s],
        )

    @pl.when(j == 0)
    def _proj():
        wf = w_ref[:H1, :]
        for b in range(B):
            ff_vmem[b] = (
                jnp.dot(f_ref[:, b, :], wf, preferred_element_type=jnp.float32)
                + bias_ref[0]
            )

        @pl.when(ti == 0)
        def _():
            wg = w_ref[H1:, :]
            for b in range(B):
                gg_vmem[b] = jnp.dot(
                    g_ref[b], wg, preferred_element_type=jnp.float32)

    @pl.when(j > 0)
    def _add():
        b = j - 1
        astep = ti * B + b
        slot = jax.lax.rem(astep, NBUF)

        @pl.when(astep >= NBUF)
        def _():
            prev = astep - NBUF
            out_copy(jax.lax.rem(prev, B), prev // B, slot).wait()

        f_len = lens_ref[0, b]
        g_len = lens_ref[1, b]

        ff = ff_vmem[b]
        gg = gg_vmem[b]
        U, V = gg.shape

        t_ids = ti * TB + jax.lax.broadcasted_iota(jnp.int32, (TB, V), 0)
        u_ids = jax.lax.broadcasted_iota(jnp.int32, (U, V), 0)
        tmask = (t_ids < f_len).astype(jnp.float32)
        umask = (u_ids < g_len).astype(jnp.float32)

        summed = ff[:, None, :] + gg[None, :, :]
        out_vmem[slot] = summed * tmask[:, None, :] * umask[None, :, :]

        out_copy(b, ti, slot).start()

        @pl.when(astep == NT * B - 1)
        def _():
            for k in range(NBUF - 1, -1, -1):
                prev = NT * B - 1 - k
                s = jax.lax.rem(prev, NBUF)
                out_copy(jax.lax.rem(prev, B), prev // B, s).wait()


def kernel(f, f_lens, g, g_lens, W, b):
    T, B, H1 = f.shape
    _, U, H2 = g.shape
    V = W.shape[1]
    NT = T // TB

    lens = jnp.stack([f_lens, g_lens]).astype(jnp.int32)
    bias2d = b.reshape(1, V)

    out = pl.pallas_call(
        functools.partial(_joint_kernel, H1=H1, B=B, NT=NT),
        grid_spec=pltpu.PrefetchScalarGridSpec(
            num_scalar_prefetch=1,
            grid=(NT, 1 + B),
            in_specs=[
                pl.BlockSpec((TB, B, H1), lambda ti, j, lens: (ti, 0, 0)),
                pl.BlockSpec((B, U, H2), lambda ti, j, lens: (0, 0, 0)),
                pl.BlockSpec((H1 + H2, V), lambda ti, j, lens: (0, 0)),
                pl.BlockSpec((1, V), lambda ti, j, lens: (0, 0)),
            ],
            out_specs=pl.BlockSpec(memory_space=pl.ANY),
            scratch_shapes=[
                pltpu.VMEM((B, TB, V), jnp.float32),
                pltpu.VMEM((B, U, V), jnp.float32),
                pltpu.VMEM((NBUF, TB, U, V), jnp.float32),
                pltpu.SemaphoreType.DMA((NBUF,)),
            ],
        ),
        out_shape=jax.ShapeDtypeStruct((B, T, U, V), jnp.float32),
    )(lens, f, g, W, bias2d)
    return (out, f_lens)
